# Initial kernel scaffold; baseline (speedup 1.0000x reference)
import jax
import jax.numpy as jnp
from jax import lax
from jax.experimental import pallas as pl
from jax.experimental.pallas import tpu as pltpu
from jax.experimental.pallas import tpu_sc as plsc

K = 64
I = 100000
B = 50
CW = 3128
ZPAD = 3136
LAM2 = 2.0 * 1e-4


def _body(y_hbm, ids_hbm, like_hbm, x_hbm, out_hbm,
          zbuf, ids_v, like_v, x_v, iidx, ycol, gcol,
          sem_in, sem_zero, sem_g, sem_s):
    wid = lax.axis_index("c") * 16 + lax.axis_index("s")
    olo = wid * CW
    ohi = jnp.minimum(olo + CW, I)
    zbase = jnp.minimum(olo, I - CW)

    h_ids = pltpu.async_copy(ids_hbm, ids_v, sem_in)
    h_like = pltpu.async_copy(like_hbm, like_v, sem_in)
    h_x = pltpu.async_copy(x_hbm, x_v, sem_in)

    zeros16 = jnp.zeros((16,), jnp.float32)

    def memset(i, c):
        zbuf[pl.ds(i * 16, 16)] = zeros16
        return c

    lax.fori_loop(0, ZPAD // 16, memset, 0)

    def fire(k, c):
        pltpu.async_copy(zbuf.at[pl.ds(0, CW)],
                         out_hbm.at[pl.ds(k * I + zbase, CW)], sem_zero)
        return c

    lax.fori_loop(0, K, fire, 0)

    h_ids.wait()
    h_like.wait()
    h_x.wait()

    def drain(k, c):
        pltpu.make_async_copy(zbuf.at[pl.ds(0, CW)],
                              out_hbm.at[pl.ds(k * I + zbase, CW)],
                              sem_zero).wait()
        return c

    lax.fori_loop(0, K, drain, 0)

    plsc.subcore_barrier()

    lane_off = lax.broadcasted_iota(jnp.int32, (16,), 0) * I

    def per_item(b, c):
        mid = ids_v[b]

        @pl.when(jnp.logical_and(mid >= olo, mid < ohi))
        def _():
            for ch in range(4):
                iidx[pl.ds(ch * 16, 16)] = lane_off + (ch * 16 * I + mid)
            pltpu.async_copy(y_hbm.at[iidx], ycol, sem_g).wait()
            pred = jnp.float32(0.0)
            for ch in range(4):
                pred = pred + jnp.sum(x_v[pl.ds(ch * 16, 16)] *
                                      ycol[pl.ds(ch * 16, 16)])
            coeff = (pred - like_v[b]) * jnp.float32(2.0 / B)
            for ch in range(4):
                gcol[pl.ds(ch * 16, 16)] = (coeff * x_v[pl.ds(ch * 16, 16)] +
                                            jnp.float32(LAM2 / B) *
                                            ycol[pl.ds(ch * 16, 16)])
            pltpu.async_copy(gcol, out_hbm.at[iidx], sem_s).wait()

        return c

    lax.fori_loop(0, B, per_item, 0)


_call = pl.kernel(
    _body,
    out_type=jax.ShapeDtypeStruct((K * I,), jnp.float32),
    mesh=plsc.VectorSubcoreMesh(core_axis_name="c", subcore_axis_name="s"),
    scratch_types=[
        pltpu.VMEM((ZPAD,), jnp.float32),
        pltpu.VMEM((64,), jnp.int32),
        pltpu.VMEM((64,), jnp.float32),
        pltpu.VMEM((64,), jnp.float32),
        pltpu.VMEM((64,), jnp.int32),
        pltpu.VMEM((64,), jnp.float32),
        pltpu.VMEM((64,), jnp.float32),
        pltpu.SemaphoreType.DMA,
        pltpu.SemaphoreType.DMA,
        pltpu.SemaphoreType.DMA,
        pltpu.SemaphoreType.DMA,
    ],
)


def kernel(global_Y, movie_ids, is_like, X):
    y_flat = global_Y.reshape(-1)
    ids64 = jnp.zeros((64,), jnp.int32).at[:B].set(movie_ids)
    like64 = jnp.zeros((64,), jnp.float32).at[:B].set(is_like)
    x64 = X.reshape(-1)
    out = _call(y_flat, ids64, like64, x64)
    return out.reshape(K, I)


# speedup vs baseline: 1.1492x; 1.1492x over previous
import jax
import jax.numpy as jnp
from jax import lax
from jax.experimental import pallas as pl
from jax.experimental.pallas import tpu as pltpu
from jax.experimental.pallas import tpu_sc as plsc

K = 64
I = 100000
B = 50
CW = 3128
ZPAD = 3136
LAM2 = 2.0 * 1e-4


def _body(y_hbm, ids_hbm, like_hbm, x_hbm, out_hbm,
          zbuf, ids_v, like_v, x_v, iidx, ycol, gcol, psum,
          sem_in, sem_zero, sem_g, sem_s):
    wid = lax.axis_index("c") * 16 + lax.axis_index("s")
    olo = wid * CW
    ohi = jnp.minimum(olo + CW, I)
    zbase = jnp.minimum(olo, I - CW)

    h_ids = pltpu.async_copy(ids_hbm, ids_v, sem_in)
    h_like = pltpu.async_copy(like_hbm, like_v, sem_in)
    h_x = pltpu.async_copy(x_hbm, x_v, sem_in)

    zeros16 = jnp.zeros((16,), jnp.float32)

    def memset(i, c):
        zbuf[pl.ds(i * 16, 16)] = zeros16
        return c

    lax.fori_loop(0, ZPAD // 16, memset, 0)

    def fire(k, c):
        pltpu.async_copy(zbuf.at[pl.ds(0, CW)],
                         out_hbm.at[pl.ds(k * I + zbase, CW)], sem_zero)
        return c

    lax.fori_loop(0, K, fire, 0)

    h_ids.wait()
    h_like.wait()
    h_x.wait()

    def drain(k, c):
        pltpu.make_async_copy(zbuf.at[pl.ds(0, CW)],
                              out_hbm.at[pl.ds(k * I + zbase, CW)],
                              sem_zero).wait()
        return c

    lax.fori_loop(0, K, drain, 0)

    plsc.subcore_barrier()

    lane_off = lax.broadcasted_iota(jnp.int32, (16,), 0) * I

    def per_item(b, c):
        mid = ids_v[pl.ds(b, 16)][0]

        @pl.when(jnp.logical_and(mid >= olo, mid < ohi))
        def _():
            for ch in range(4):
                iidx[pl.ds(ch * 16, 16)] = lane_off + (ch * 16 * I + mid)
            pltpu.async_copy(y_hbm.at[iidx], ycol, sem_g).wait()
            psum[pl.ds(0, 16)] = zeros16
            zidx = jnp.zeros((16,), jnp.int32)
            for ch in range(4):
                plsc.addupdate_scatter(
                    psum, [zidx],
                    x_v[pl.ds(ch * 16, 16)] * ycol[pl.ds(ch * 16, 16)])
            pred = psum[pl.ds(0, 16)][0]
            coeff = (pred - like_v[pl.ds(b, 16)][0]) * jnp.float32(2.0 / B)
            for ch in range(4):
                gcol[pl.ds(ch * 16, 16)] = (coeff * x_v[pl.ds(ch * 16, 16)] +
                                            jnp.float32(LAM2 / B) *
                                            ycol[pl.ds(ch * 16, 16)])
            pltpu.async_copy(gcol, out_hbm.at[iidx], sem_s).wait()

        return c

    lax.fori_loop(0, B, per_item, 0)


_call = pl.kernel(
    _body,
    out_type=jax.ShapeDtypeStruct((K * I,), jnp.float32),
    mesh=plsc.VectorSubcoreMesh(core_axis_name="c", subcore_axis_name="s"),
    compiler_params=pltpu.CompilerParams(needs_layout_passes=False),
    scratch_types=[
        pltpu.VMEM((ZPAD,), jnp.float32),
        pltpu.VMEM((80,), jnp.int32),
        pltpu.VMEM((80,), jnp.float32),
        pltpu.VMEM((64,), jnp.float32),
        pltpu.VMEM((64,), jnp.int32),
        pltpu.VMEM((64,), jnp.float32),
        pltpu.VMEM((64,), jnp.float32),
        pltpu.VMEM((16,), jnp.float32),
        pltpu.SemaphoreType.DMA,
        pltpu.SemaphoreType.DMA,
        pltpu.SemaphoreType.DMA,
        pltpu.SemaphoreType.DMA,
    ],
)


def kernel(global_Y, movie_ids, is_like, X):
    y_flat = global_Y.reshape(-1)
    ids80 = jnp.full((80,), I, jnp.int32).at[:B].set(movie_ids)
    like80 = jnp.zeros((80,), jnp.float32).at[:B].set(is_like)
    x64 = X.reshape(-1)
    out = _call(y_flat, ids80, like80, x64)
    return out.reshape(K, I)


# speedup vs baseline: 1.1562x; 1.0061x over previous
import jax
import jax.numpy as jnp
from jax import lax
from jax.experimental import pallas as pl
from jax.experimental.pallas import tpu as pltpu
from jax.experimental.pallas import tpu_sc as plsc

K = 64
I = 100000
B = 50
CW = 3128
ZPAD = 3136
LAM2 = 2.0 * 1e-4


def _body(y_hbm, ids_hbm, like_hbm, x_hbm, out_hbm,
          zbuf, ids_v, like_v, x_v, iidx, ycol, gcol, psum,
          sem_in, sem_zero, sem_g, sem_s):
    wid = lax.axis_index("c") * 16 + lax.axis_index("s")
    olo = wid * CW
    ohi = jnp.minimum(olo + CW, I)
    zbase = jnp.minimum(olo, I - CW)

    h_ids = pltpu.async_copy(ids_hbm, ids_v.at[pl.ds(0, B)], sem_in)
    h_like = pltpu.async_copy(like_hbm, like_v.at[pl.ds(0, B)], sem_in)
    h_x = pltpu.async_copy(x_hbm, x_v, sem_in)

    zeros16 = jnp.zeros((16,), jnp.float32)

    def memset(i, c):
        zbuf[pl.ds(i * 16, 16)] = zeros16
        return c

    lax.fori_loop(0, ZPAD // 16, memset, 0)

    def fire(k, c):
        pltpu.async_copy(zbuf.at[pl.ds(0, CW)],
                         out_hbm.at[pl.ds(k * I + zbase, CW)], sem_zero)
        return c

    lax.fori_loop(0, K, fire, 0)

    h_ids.wait()
    h_like.wait()
    h_x.wait()

    def drain(k, c):
        pltpu.make_async_copy(zbuf.at[pl.ds(0, CW)],
                              out_hbm.at[pl.ds(k * I + zbase, CW)],
                              sem_zero).wait()
        return c

    lax.fori_loop(0, K, drain, 0)

    plsc.subcore_barrier()

    lane_off = lax.broadcasted_iota(jnp.int32, (16,), 0) * I

    def per_item(b, c):
        mid = ids_v[pl.ds(b, 16)][0]

        @pl.when(jnp.logical_and(mid >= olo, mid < ohi))
        def _():
            for ch in range(4):
                iidx[pl.ds(ch * 16, 16)] = lane_off + (ch * 16 * I + mid)
            pltpu.async_copy(y_hbm.at[iidx], ycol, sem_g).wait()
            psum[pl.ds(0, 16)] = zeros16
            zidx = jnp.zeros((16,), jnp.int32)
            for ch in range(4):
                plsc.addupdate_scatter(
                    psum, [zidx],
                    x_v[pl.ds(ch * 16, 16)] * ycol[pl.ds(ch * 16, 16)])
            pred = psum[pl.ds(0, 16)][0]
            coeff = (pred - like_v[pl.ds(b, 16)][0]) * jnp.float32(2.0 / B)
            for ch in range(4):
                gcol[pl.ds(ch * 16, 16)] = (coeff * x_v[pl.ds(ch * 16, 16)] +
                                            jnp.float32(LAM2 / B) *
                                            ycol[pl.ds(ch * 16, 16)])
            pltpu.async_copy(gcol, out_hbm.at[iidx], sem_s).wait()

        return c

    lax.fori_loop(0, B, per_item, 0)


_call = pl.kernel(
    _body,
    out_type=jax.ShapeDtypeStruct((K * I,), jnp.float32),
    mesh=plsc.VectorSubcoreMesh(core_axis_name="c", subcore_axis_name="s"),
    compiler_params=pltpu.CompilerParams(needs_layout_passes=False),
    scratch_types=[
        pltpu.VMEM((ZPAD,), jnp.float32),
        pltpu.VMEM((80,), jnp.int32),
        pltpu.VMEM((80,), jnp.float32),
        pltpu.VMEM((64,), jnp.float32),
        pltpu.VMEM((64,), jnp.int32),
        pltpu.VMEM((64,), jnp.float32),
        pltpu.VMEM((64,), jnp.float32),
        pltpu.VMEM((16,), jnp.float32),
        pltpu.SemaphoreType.DMA,
        pltpu.SemaphoreType.DMA,
        pltpu.SemaphoreType.DMA,
        pltpu.SemaphoreType.DMA,
    ],
)


def kernel(global_Y, movie_ids, is_like, X):
    out = _call(global_Y.reshape(-1), movie_ids, is_like, X.reshape(-1))
    return out.reshape(K, I)


# speedup vs baseline: 3.1295x; 2.7067x over previous
import jax
import jax.numpy as jnp
from jax import lax
from jax.experimental import pallas as pl
from jax.experimental.pallas import tpu as pltpu
from jax.experimental.pallas import tpu_sc as plsc

K = 64
I = 100000
B = 50
CW = 3200
TAILW = 896
LAM2 = 2.0 * 1e-4


def _body(y_hbm, ids_hbm, like_hbm, x_hbm, out_hbm,
          zbuf, ids_v, like_v, x_v, yblk, oblk, psum,
          sem_in, sem_zero, sem_g, sem_o, sem_s):
    wid = lax.axis_index("c") * 16 + lax.axis_index("s")
    olo = wid * CW
    ohi = jnp.minimum(olo + CW, I)

    h_ids = pltpu.async_copy(ids_hbm, ids_v.at[pl.ds(0, B)], sem_in)
    h_like = pltpu.async_copy(like_hbm, like_v.at[pl.ds(0, B)], sem_in)
    h_x = pltpu.async_copy(x_hbm, x_v, sem_in)

    zeros16 = jnp.zeros((16,), jnp.float32)

    def memset(i, c):
        for r in range(8):
            zbuf[r, pl.ds(i * 16, 16)] = zeros16
        return c

    lax.fori_loop(0, CW // 16, memset, 0)

    def zero_fire(base, width):
        for r in range(8):
            pltpu.async_copy(zbuf.at[pl.ds(0, 8), pl.ds(0, width)],
                             out_hbm.at[pl.ds(r * 8, 8), pl.ds(base, width)],
                             sem_zero)

    def zero_drain(base, width):
        for r in range(8):
            pltpu.make_async_copy(
                zbuf.at[pl.ds(0, 8), pl.ds(0, width)],
                out_hbm.at[pl.ds(r * 8, 8), pl.ds(base, width)],
                sem_zero).wait()

    @pl.when(wid < 31)
    def _():
        zero_fire(pl.multiple_of(wid * CW, 128), CW)

    @pl.when(wid == 31)
    def _():
        zero_fire(pl.multiple_of(wid * CW, 128), TAILW)

    h_ids.wait()
    h_like.wait()
    h_x.wait()

    @pl.when(wid < 31)
    def _():
        zero_drain(pl.multiple_of(wid * CW, 128), CW)

    @pl.when(wid == 31)
    def _():
        zero_drain(pl.multiple_of(wid * CW, 128), TAILW)

    plsc.subcore_barrier()

    iota = lax.broadcasted_iota(jnp.int32, (16,), 0)

    def per_item(b, c):
        mid = ids_v[pl.ds(b, 16)][0]

        @pl.when(jnp.logical_and(mid >= olo, mid < ohi))
        def _():
            col0 = pl.multiple_of((mid // 128) * 128, 128)
            lcol = mid - col0
            cols = lcol + jnp.zeros((16,), jnp.int32)
            pltpu.async_copy(y_hbm.at[pl.ds(0, K), pl.ds(col0, 128)],
                             yblk, sem_g).wait()
            ycs = [plsc.load_gather(yblk, [iota + 16 * ch, cols])
                   for ch in range(4)]
            xcs = [x_v[pl.ds(16 * ch, 16)] for ch in range(4)]
            psum[pl.ds(0, 16)] = zeros16
            zidx = jnp.zeros((16,), jnp.int32)
            for ch in range(4):
                plsc.addupdate_scatter(psum, [zidx], xcs[ch] * ycs[ch])
            pred = psum[pl.ds(0, 16)][0]
            coeff = (pred - like_v[pl.ds(b, 16)][0]) * jnp.float32(2.0 / B)
            h_o = pltpu.async_copy(out_hbm.at[pl.ds(0, K), pl.ds(col0, 128)],
                                   oblk, sem_o)
            h_o.wait()
            for ch in range(4):
                gc = coeff * xcs[ch] + jnp.float32(LAM2 / B) * ycs[ch]
                plsc.store_scatter(oblk, [iota + 16 * ch, cols], gc)
            pltpu.async_copy(oblk,
                             out_hbm.at[pl.ds(0, K), pl.ds(col0, 128)],
                             sem_s).wait()

        return c

    lax.fori_loop(0, B, per_item, 0)


_call = pl.kernel(
    _body,
    out_type=jax.ShapeDtypeStruct((K, I), jnp.float32),
    mesh=plsc.VectorSubcoreMesh(core_axis_name="c", subcore_axis_name="s"),
    compiler_params=pltpu.CompilerParams(needs_layout_passes=False,
                                         use_tc_tiling_on_sc=True),
    scratch_types=[
        pltpu.VMEM((8, CW), jnp.float32),
        pltpu.VMEM((80,), jnp.int32),
        pltpu.VMEM((80,), jnp.float32),
        pltpu.VMEM((64,), jnp.float32),
        pltpu.VMEM((K, 128), jnp.float32),
        pltpu.VMEM((K, 128), jnp.float32),
        pltpu.VMEM((16,), jnp.float32),
        pltpu.SemaphoreType.DMA,
        pltpu.SemaphoreType.DMA,
        pltpu.SemaphoreType.DMA,
        pltpu.SemaphoreType.DMA,
        pltpu.SemaphoreType.DMA,
    ],
)


def kernel(global_Y, movie_ids, is_like, X):
    return _call(global_Y, movie_ids, is_like, X.reshape(-1))


# speedup vs baseline: 3.2258x; 1.0308x over previous
import jax
import jax.numpy as jnp
from jax import lax
from jax.experimental import pallas as pl
from jax.experimental.pallas import tpu as pltpu
from jax.experimental.pallas import tpu_sc as plsc

K = 64
I = 100000
B = 50
CW = 3200
TAILW = 896
LAM2 = 2.0 * 1e-4


def _body(y_hbm, ids_hbm, like_hbm, x_hbm, out_hbm,
          zbuf, ids_v, like_v, x_v, yblk, oblk, psum, gbuf,
          sem_in, sem_zero, sem_g, sem_o, sem_s):
    wid = lax.axis_index("c") * 16 + lax.axis_index("s")
    olo = wid * CW
    ohi = jnp.minimum(olo + CW, I)

    h_ids = pltpu.async_copy(ids_hbm, ids_v.at[pl.ds(0, B)], sem_in)
    h_like = pltpu.async_copy(like_hbm, like_v.at[pl.ds(0, B)], sem_in)
    h_x = pltpu.async_copy(x_hbm, x_v, sem_in)

    zeros16 = jnp.zeros((16,), jnp.float32)
    zidx = jnp.zeros((16,), jnp.int32)
    iota = lax.broadcasted_iota(jnp.int32, (16,), 0)

    def memset(i, c):
        for r in range(8):
            zbuf[r, pl.ds(i * 16, 16)] = zeros16
        return c

    lax.fori_loop(0, CW // 16, memset, 0)

    def zero_fire(base, width):
        for r in range(8):
            pltpu.async_copy(zbuf.at[pl.ds(0, 8), pl.ds(0, width)],
                             out_hbm.at[pl.ds(r * 8, 8), pl.ds(base, width)],
                             sem_zero)

    def zero_drain(base, width):
        for r in range(8):
            pltpu.make_async_copy(
                zbuf.at[pl.ds(0, 8), pl.ds(0, width)],
                out_hbm.at[pl.ds(r * 8, 8), pl.ds(base, width)],
                sem_zero).wait()

    @pl.when(wid < 31)
    def _():
        zero_fire(pl.multiple_of(wid * CW, 128), CW)

    @pl.when(wid == 31)
    def _():
        zero_fire(pl.multiple_of(wid * CW, 128), TAILW)

    h_ids.wait()
    h_like.wait()
    h_x.wait()

    xcs = [plsc.load_gather(x_v, [zidx, iota + 16 * ch]) for ch in range(4)]

    def per_item_compute(b, c):
        mid = ids_v[pl.ds(b, 16)][0]

        @pl.when(jnp.logical_and(mid >= olo, mid < ohi))
        def _():
            col0 = pl.multiple_of((mid // 128) * 128, 128)
            cols = (mid - col0) + zidx
            pltpu.async_copy(y_hbm.at[pl.ds(0, K), pl.ds(col0, 128)],
                             yblk, sem_g).wait()
            ycs = [plsc.load_gather(yblk, [iota + 16 * ch, cols])
                   for ch in range(4)]
            psum[pl.ds(0, 16)] = zeros16
            for ch in range(4):
                plsc.addupdate_scatter(psum, [zidx], xcs[ch] * ycs[ch])
            pred = psum[pl.ds(0, 16)][0]
            coeff = (pred - like_v[pl.ds(b, 16)][0]) * jnp.float32(2.0 / B)
            for ch in range(4):
                gbuf[pl.ds(b * 64 + 16 * ch, 16)] = (
                    coeff * xcs[ch] + jnp.float32(LAM2 / B) * ycs[ch])

        return c

    lax.fori_loop(0, B, per_item_compute, 0)

    @pl.when(wid < 31)
    def _():
        zero_drain(pl.multiple_of(wid * CW, 128), CW)

    @pl.when(wid == 31)
    def _():
        zero_drain(pl.multiple_of(wid * CW, 128), TAILW)

    def per_item_scatter(b, c):
        mid = ids_v[pl.ds(b, 16)][0]

        @pl.when(jnp.logical_and(mid >= olo, mid < ohi))
        def _():
            col0 = pl.multiple_of((mid // 128) * 128, 128)
            cols = (mid - col0) + zidx
            pltpu.async_copy(out_hbm.at[pl.ds(0, K), pl.ds(col0, 128)],
                             oblk, sem_o).wait()
            for ch in range(4):
                plsc.store_scatter(oblk, [iota + 16 * ch, cols],
                                   gbuf[pl.ds(b * 64 + 16 * ch, 16)])
            pltpu.async_copy(oblk,
                             out_hbm.at[pl.ds(0, K), pl.ds(col0, 128)],
                             sem_s).wait()

        return c

    lax.fori_loop(0, B, per_item_scatter, 0)


_call = pl.kernel(
    _body,
    out_type=jax.ShapeDtypeStruct((K, I), jnp.float32),
    mesh=plsc.VectorSubcoreMesh(core_axis_name="c", subcore_axis_name="s"),
    compiler_params=pltpu.CompilerParams(needs_layout_passes=False,
                                         use_tc_tiling_on_sc=True),
    scratch_types=[
        pltpu.VMEM((8, CW), jnp.float32),
        pltpu.VMEM((80,), jnp.int32),
        pltpu.VMEM((80,), jnp.float32),
        pltpu.VMEM((1, 64), jnp.float32),
        pltpu.VMEM((K, 128), jnp.float32),
        pltpu.VMEM((K, 128), jnp.float32),
        pltpu.VMEM((16,), jnp.float32),
        pltpu.VMEM((B * 64,), jnp.float32),
        pltpu.SemaphoreType.DMA,
        pltpu.SemaphoreType.DMA,
        pltpu.SemaphoreType.DMA,
        pltpu.SemaphoreType.DMA,
        pltpu.SemaphoreType.DMA,
    ],
)


def kernel(global_Y, movie_ids, is_like, X):
    return _call(global_Y, movie_ids, is_like, X)
